# baseline (speedup 1.0000x reference)
import functools

import jax
import jax.numpy as jnp
from jax import lax
from jax.experimental import pallas as pl
from jax.experimental.pallas import tpu as pltpu
from jax.experimental.pallas import tpu_sc as plsc

EW = 128
MB = 2000


def _dinv16(degS, n):
    def body(d_ref, out_ref):
        out_ref[...] = 1.0 / jnp.sqrt(1.0 + d_ref[0, :, :16] + d_ref[1, :, :16])

    return pl.pallas_call(
        body,
        grid=(n // MB,),
        in_specs=[pl.BlockSpec((2, MB, 128), lambda m: (0, m, 0))],
        out_specs=pl.BlockSpec((MB, 16), lambda m: (m, 0)),
        out_shape=jax.ShapeDtypeStruct((n, 16), jnp.float32),
    )(degS)


def _sc_deg(dst2, zeros128, ones128):
    nrows, ew = dst2.shape
    rows_per_tile = nrows // 32
    n_pad = zeros128.shape[0]
    rpt_n = n_pad // 16
    mesh = plsc.VectorSubcoreMesh(core_axis_name="c", subcore_axis_name="s",
                                  num_cores=2, num_subcores=16)

    @functools.partial(
        pl.kernel,
        out_type=jax.ShapeDtypeStruct((2, n_pad, 128), jnp.float32),
        mesh=mesh,
        scratch_types=[
            pltpu.VMEM((rows_per_tile, ew), jnp.int32),
            pltpu.VMEM((ew, 128), jnp.float32),
            pltpu.VMEM_SHARED((n_pad, 128), jnp.float32),
        ],
    )
    def k(dst_hbm, z_hbm, ones_hbm, out_hbm, dst_v, ones_v, s_sh):
        cid = lax.axis_index("c")
        sid = lax.axis_index("s")
        er0 = (cid * 16 + sid) * rows_per_tile
        pltpu.sync_copy(dst_hbm.at[pl.ds(er0, rows_per_tile)], dst_v)
        pltpu.sync_copy(ones_hbm, ones_v)
        nb = sid * rpt_n
        pltpu.sync_copy(z_hbm.at[pl.ds(nb, rpt_n)], s_sh.at[pl.ds(nb, rpt_n)])
        plsc.subcore_barrier()

        def body(j, carry):
            pltpu.sync_copy(ones_v, s_sh.at[dst_v.at[j]], add=True)
            return carry

        lax.fori_loop(0, rows_per_tile, body, 0)
        plsc.subcore_barrier()
        pltpu.sync_copy(s_sh.at[pl.ds(nb, rpt_n)],
                        out_hbm.at[cid, pl.ds(nb, rpt_n)])

    return k(dst2, zeros128, ones128)


def _matmul_g(h, W, dinv16, n_pad):
    n, fin = h.shape
    fout = W.shape[1]
    cin, cout = fin // 128, fout // 128

    def body(h_ref, w_ref, dinv_ref, out_ref):
        ci = pl.program_id(2)

        @pl.when(ci == 0)
        def _():
            out_ref[...] = jnp.zeros(out_ref.shape, out_ref.dtype)

        out_ref[0] += jnp.dot(h_ref[...], w_ref[...],
                              preferred_element_type=jnp.float32)

        @pl.when(ci == cin - 1)
        def _():
            out_ref[0] = out_ref[0] * dinv_ref[:, :1]

    return pl.pallas_call(
        body,
        grid=(n // MB, cout, cin),
        in_specs=[
            pl.BlockSpec((MB, 128), lambda m, co, ci: (m, ci)),
            pl.BlockSpec((128, 128), lambda m, co, ci: (ci, co)),
            pl.BlockSpec((MB, 16), lambda m, co, ci: (m, 0)),
        ],
        out_specs=pl.BlockSpec((1, MB, 128), lambda m, co, ci: (co, m, 0)),
        out_shape=jax.ShapeDtypeStruct((cout, n_pad, 128), jnp.float32),
        compiler_params=pltpu.CompilerParams(
            dimension_semantics=("parallel", "parallel", "arbitrary")),
    )(h, W, dinv16)


def _activate(S, dinv16, b2d, n):
    c = S.shape[0]

    def body(s_ref, dinv_ref, b_ref, out_ref):
        out_ref[...] = jnp.tanh(s_ref[0] * dinv_ref[:, :1] + b_ref[0])

    return pl.pallas_call(
        body,
        grid=(n // MB, c),
        in_specs=[
            pl.BlockSpec((1, MB, 128), lambda m, c_: (c_, m, 0)),
            pl.BlockSpec((MB, 16), lambda m, c_: (m, 0)),
            pl.BlockSpec((1, 1, 128), lambda m, c_: (c_, 0, 0)),
        ],
        out_specs=pl.BlockSpec((MB, 128), lambda m, c_: (m, c_)),
        out_shape=jax.ShapeDtypeStruct((n, c * 128), jnp.float32),
    )(S, dinv16, b2d)


def _sc_edge(g, src2, dst2):
    c, n_pad, _ = g.shape
    nrows, ew = src2.shape
    rows_per_tile = nrows // 16
    rpt_n = n_pad // 16
    mesh = plsc.VectorSubcoreMesh(core_axis_name="c", subcore_axis_name="s", num_cores=2, num_subcores=16)

    @functools.partial(
        pl.kernel,
        out_type=jax.ShapeDtypeStruct((c, n_pad, 128), jnp.float32),
        mesh=mesh,
        scratch_types=[
            pltpu.VMEM((rows_per_tile, ew), jnp.int32),
            pltpu.VMEM((rows_per_tile, ew), jnp.int32),
            pltpu.VMEM((ew, 128), jnp.float32),
            pltpu.VMEM_SHARED((n_pad, 128), jnp.float32),
        ],
    )
    def k(g_hbm, src_hbm, dst_hbm, out_hbm, src_v, dst_v, rows_v, s_sh):
        cid = lax.axis_index("c")
        sid = lax.axis_index("s")
        er0 = sid * rows_per_tile
        pltpu.sync_copy(src_hbm.at[pl.ds(er0, rows_per_tile)], src_v)
        pltpu.sync_copy(dst_hbm.at[pl.ds(er0, rows_per_tile)], dst_v)
        nb = sid * rpt_n
        for chunk in range(c):

            @pl.when(chunk % 2 == cid)
            def _(chunk=chunk):
                g_c = g_hbm.at[chunk]
                pltpu.sync_copy(g_c.at[pl.ds(nb, rpt_n)],
                                s_sh.at[pl.ds(nb, rpt_n)])
                plsc.subcore_barrier()

                def body(j, carry):
                    pltpu.sync_copy(g_c.at[src_v.at[j]], rows_v)
                    pltpu.sync_copy(rows_v, s_sh.at[dst_v.at[j]], add=True)
                    return carry

                lax.fori_loop(0, rows_per_tile, body, 0)
                plsc.subcore_barrier()
                pltpu.sync_copy(s_sh.at[pl.ds(nb, rpt_n)],
                                out_hbm.at[chunk, pl.ds(nb, rpt_n)])
                plsc.subcore_barrier()

    return k(g, src2, dst2)


def kernel(x, edge_index, W1, b1, W2, b2, W3, b3):
    n = x.shape[0]
    e = edge_index.shape[1]
    n_pad = -(-n // 128) * 128
    e_pad = -(-e // (32 * EW)) * (32 * EW)
    pad = jnp.full((e_pad - e,), n, jnp.int32)
    src2 = jnp.concatenate([edge_index[0], pad]).reshape(e_pad // EW, EW)
    dst2 = jnp.concatenate([edge_index[1], pad]).reshape(e_pad // EW, EW)

    degS = _sc_deg(dst2, jnp.zeros((n_pad, 128), jnp.float32),
                   jnp.ones((EW, 128), jnp.float32))
    dinv16 = _dinv16(degS, n)

    h = x
    for W, b in ((W1, b1), (W2, b2), (W3, b3)):
        g = _matmul_g(h, W, dinv16, n_pad)
        S = _sc_edge(g, src2, dst2)
        h = _activate(S, dinv16, b.reshape(-1, 1, 128), n)
    return h


# speedup vs baseline: 1.7159x; 1.7159x over previous
import functools

import jax
import jax.numpy as jnp
from jax import lax
from jax.experimental import pallas as pl
from jax.experimental.pallas import tpu as pltpu
from jax.experimental.pallas import tpu_sc as plsc

EW = 125
MB = 2000


def _dinv16(degS, n):
    def body(d_ref, out_ref):
        out_ref[...] = 1.0 / jnp.sqrt(1.0 + d_ref[0, :, :16] + d_ref[1, :, :16])

    return pl.pallas_call(
        body,
        grid=(n // MB,),
        in_specs=[pl.BlockSpec((2, MB, 128), lambda m: (0, m, 0))],
        out_specs=pl.BlockSpec((MB, 16), lambda m: (m, 0)),
        out_shape=jax.ShapeDtypeStruct((n, 16), jnp.float32),
    )(degS)


def _sc_deg(dst2, zeros128, ones128):
    nrows, ew = dst2.shape
    rows_per_tile = nrows // 32
    n_pad = zeros128.shape[0]
    rpt_n = n_pad // 16
    mesh = plsc.VectorSubcoreMesh(core_axis_name="c", subcore_axis_name="s",
                                  num_cores=2, num_subcores=16)

    @functools.partial(
        pl.kernel,
        out_type=jax.ShapeDtypeStruct((2, n_pad, 128), jnp.float32),
        mesh=mesh,
        scratch_types=[
            pltpu.VMEM((rows_per_tile, ew), jnp.int32),
            pltpu.VMEM((ew, 128), jnp.float32),
            pltpu.VMEM_SHARED((n_pad, 128), jnp.float32),
        ],
    )
    def k(dst_hbm, z_hbm, ones_hbm, out_hbm, dst_v, ones_v, s_sh):
        cid = lax.axis_index("c")
        sid = lax.axis_index("s")
        er0 = (cid * 16 + sid) * rows_per_tile
        pltpu.sync_copy(dst_hbm.at[pl.ds(er0, rows_per_tile)], dst_v)
        pltpu.sync_copy(ones_hbm, ones_v)
        nb = sid * rpt_n
        pltpu.sync_copy(z_hbm.at[pl.ds(nb, rpt_n)], s_sh.at[pl.ds(nb, rpt_n)])
        plsc.subcore_barrier()

        def body(j, carry):
            pltpu.sync_copy(ones_v, s_sh.at[dst_v.at[j]], add=True)
            return carry

        lax.fori_loop(0, rows_per_tile, body, 0)
        plsc.subcore_barrier()
        pltpu.sync_copy(s_sh.at[pl.ds(nb, rpt_n)],
                        out_hbm.at[cid, pl.ds(nb, rpt_n)])

    return k(dst2, zeros128, ones128)


def _matmul_g(h, W, dinv16, n_pad):
    n, fin = h.shape
    fout = W.shape[1]
    cin, cout = fin // 128, fout // 128

    def body(h_ref, w_ref, dinv_ref, out_ref):
        ci = pl.program_id(2)

        @pl.when(ci == 0)
        def _():
            out_ref[...] = jnp.zeros(out_ref.shape, out_ref.dtype)

        out_ref[0] += jnp.dot(h_ref[...], w_ref[...],
                              preferred_element_type=jnp.float32)

        @pl.when(ci == cin - 1)
        def _():
            out_ref[0] = out_ref[0] * dinv_ref[:, :1]

    return pl.pallas_call(
        body,
        grid=(n // MB, cout, cin),
        in_specs=[
            pl.BlockSpec((MB, 128), lambda m, co, ci: (m, ci)),
            pl.BlockSpec((128, 128), lambda m, co, ci: (ci, co)),
            pl.BlockSpec((MB, 16), lambda m, co, ci: (m, 0)),
        ],
        out_specs=pl.BlockSpec((1, MB, 128), lambda m, co, ci: (co, m, 0)),
        out_shape=jax.ShapeDtypeStruct((cout, n_pad, 128), jnp.float32),
        compiler_params=pltpu.CompilerParams(
            dimension_semantics=("parallel", "parallel", "arbitrary")),
    )(h, W, dinv16)


def _activate(S, dinv16, b2d, n):
    c = S.shape[0]

    def body(s_ref, dinv_ref, b_ref, out_ref):
        out_ref[...] = jnp.tanh(s_ref[0] * dinv_ref[:, :1] + b_ref[0])

    return pl.pallas_call(
        body,
        grid=(n // MB, c),
        in_specs=[
            pl.BlockSpec((1, MB, 128), lambda m, c_: (c_, m, 0)),
            pl.BlockSpec((MB, 16), lambda m, c_: (m, 0)),
            pl.BlockSpec((1, 1, 128), lambda m, c_: (c_, 0, 0)),
        ],
        out_specs=pl.BlockSpec((MB, 128), lambda m, c_: (m, c_)),
        out_shape=jax.ShapeDtypeStruct((n, c * 128), jnp.float32),
    )(S, dinv16, b2d)


def _sc_edge(g, src2, dst2):
    c, n_pad, _ = g.shape
    nrows, ew = src2.shape
    rows_per_tile = nrows // 16
    rpt_n = n_pad // 16
    mesh = plsc.VectorSubcoreMesh(core_axis_name="c", subcore_axis_name="s", num_cores=2, num_subcores=16)

    @functools.partial(
        pl.kernel,
        out_type=jax.ShapeDtypeStruct((c, n_pad, 128), jnp.float32),
        mesh=mesh,
        scratch_types=[
            pltpu.VMEM((rows_per_tile, ew), jnp.int32),
            pltpu.VMEM((rows_per_tile, ew), jnp.int32),
            pltpu.VMEM((ew, 128), jnp.float32),
            pltpu.VMEM_SHARED((n_pad, 128), jnp.float32),
        ],
    )
    def k(g_hbm, src_hbm, dst_hbm, out_hbm, src_v, dst_v, rows_v, s_sh):
        cid = lax.axis_index("c")
        sid = lax.axis_index("s")
        er0 = sid * rows_per_tile
        pltpu.sync_copy(src_hbm.at[pl.ds(er0, rows_per_tile)], src_v)
        pltpu.sync_copy(dst_hbm.at[pl.ds(er0, rows_per_tile)], dst_v)
        nb = sid * rpt_n
        for chunk in range(c):

            @pl.when(chunk % 2 == cid)
            def _(chunk=chunk):
                g_c = g_hbm.at[chunk]
                pltpu.sync_copy(g_c.at[pl.ds(nb, rpt_n)],
                                s_sh.at[pl.ds(nb, rpt_n)])
                plsc.subcore_barrier()

                def body(j, carry):
                    pltpu.sync_copy(g_c.at[src_v.at[j]], rows_v)
                    pltpu.sync_copy(rows_v, s_sh.at[dst_v.at[j]], add=True)
                    return carry

                lax.fori_loop(0, rows_per_tile, body, 0)
                plsc.subcore_barrier()
                pltpu.sync_copy(s_sh.at[pl.ds(nb, rpt_n)],
                                out_hbm.at[chunk, pl.ds(nb, rpt_n)])
                plsc.subcore_barrier()

    return k(g, src2, dst2)


def kernel(x, edge_index, W1, b1, W2, b2, W3, b3):
    n = x.shape[0]
    e = edge_index.shape[1]
    n_pad = -(-n // 128) * 128
    src2 = edge_index[0].reshape(e // EW, EW)
    dst2 = edge_index[1].reshape(e // EW, EW)

    degS = _sc_deg(dst2, jnp.zeros((n_pad, 128), jnp.float32),
                   jnp.ones((EW, 128), jnp.float32))
    dinv16 = _dinv16(degS, n)

    h = x
    for W, b in ((W1, b1), (W2, b2), (W3, b3)):
        g = _matmul_g(h, W, dinv16, n_pad)
        S = _sc_edge(g, src2, dst2)
        h = _activate(S, dinv16, b.reshape(-1, 1, 128), n)
    return h


# speedup vs baseline: 1.9374x; 1.1291x over previous
import functools

import jax
import jax.numpy as jnp
from jax import lax
from jax.experimental import pallas as pl
from jax.experimental.pallas import tpu as pltpu
from jax.experimental.pallas import tpu_sc as plsc

EW = 125
MB = 2000


def _dinv16(degS, n):
    def body(d_ref, out_ref):
        out_ref[...] = 1.0 / jnp.sqrt(1.0 + d_ref[0, :, :16] + d_ref[1, :, :16])

    return pl.pallas_call(
        body,
        grid=(n // MB,),
        in_specs=[pl.BlockSpec((2, MB, 128), lambda m: (0, m, 0))],
        out_specs=pl.BlockSpec((MB, 16), lambda m: (m, 0)),
        out_shape=jax.ShapeDtypeStruct((n, 16), jnp.float32),
    )(degS)


def _sc_deg(dst2, zeros128, ones128):
    nrows, ew = dst2.shape
    rows_per_tile = nrows // 32
    n_pad = zeros128.shape[0]
    rpt_n = n_pad // 16
    mesh = plsc.VectorSubcoreMesh(core_axis_name="c", subcore_axis_name="s",
                                  num_cores=2, num_subcores=16)

    @functools.partial(
        pl.kernel,
        out_type=jax.ShapeDtypeStruct((2, n_pad, 128), jnp.float32),
        mesh=mesh,
        scratch_types=[
            pltpu.VMEM((rows_per_tile, ew), jnp.int32),
            pltpu.VMEM((ew, 128), jnp.float32),
            pltpu.VMEM_SHARED((n_pad, 128), jnp.float32),
        ],
    )
    def k(dst_hbm, z_hbm, ones_hbm, out_hbm, dst_v, ones_v, s_sh):
        cid = lax.axis_index("c")
        sid = lax.axis_index("s")
        er0 = (cid * 16 + sid) * rows_per_tile
        pltpu.sync_copy(dst_hbm.at[pl.ds(er0, rows_per_tile)], dst_v)
        pltpu.sync_copy(ones_hbm, ones_v)
        nb = sid * rpt_n
        pltpu.sync_copy(z_hbm.at[pl.ds(nb, rpt_n)], s_sh.at[pl.ds(nb, rpt_n)])
        plsc.subcore_barrier()

        def body(j, carry):
            pltpu.sync_copy(ones_v, s_sh.at[dst_v.at[j]], add=True)
            return carry

        lax.fori_loop(0, rows_per_tile, body, 0)
        plsc.subcore_barrier()
        pltpu.sync_copy(s_sh.at[pl.ds(nb, rpt_n)],
                        out_hbm.at[cid, pl.ds(nb, rpt_n)])

    return k(dst2, zeros128, ones128)


def _matmul_g(h, W, dinv16, n_pad):
    n, fin = h.shape
    fout = W.shape[1]
    cin, cout = fin // 128, fout // 128

    def body(h_ref, w_ref, dinv_ref, out_ref):
        r = jnp.dot(h_ref[...], w_ref[...],
                    preferred_element_type=jnp.float32) * dinv_ref[:, :1]
        out_ref[0] = r[:, :128]
        out_ref[1] = r[:, 128:]

    return pl.pallas_call(
        body,
        grid=(n // MB, cout // 2),
        in_specs=[
            pl.BlockSpec((MB, fin), lambda m, co: (m, 0)),
            pl.BlockSpec((fin, 256), lambda m, co: (0, co)),
            pl.BlockSpec((MB, 16), lambda m, co: (m, 0)),
        ],
        out_specs=pl.BlockSpec((2, MB, 128), lambda m, co: (co, m, 0)),
        out_shape=jax.ShapeDtypeStruct((cout, n_pad, 128), jnp.float32),
        compiler_params=pltpu.CompilerParams(
            dimension_semantics=("parallel", "parallel")),
    )(h, W, dinv16)


def _activate(S, dinv16, b2d, n):
    c = S.shape[0]

    def body(s_ref, dinv_ref, b_ref, out_ref):
        dinv = dinv_ref[:, :1]
        for ci in range(c):
            out_ref[:, ci * 128:(ci + 1) * 128] = jnp.tanh(
                s_ref[ci] * dinv + b_ref[ci])

    return pl.pallas_call(
        body,
        grid=(n // MB,),
        in_specs=[
            pl.BlockSpec((c, MB, 128), lambda m: (0, m, 0)),
            pl.BlockSpec((MB, 16), lambda m: (m, 0)),
            pl.BlockSpec((c, 1, 128), lambda m: (0, 0, 0)),
        ],
        out_specs=pl.BlockSpec((MB, c * 128), lambda m: (m, 0)),
        out_shape=jax.ShapeDtypeStruct((n, c * 128), jnp.float32),
    )(S, dinv16, b2d)


def _sc_edge(g, src2, dst2):
    c, n_pad, _ = g.shape
    nrows, ew = src2.shape
    rows_per_tile = nrows // 16
    rpt_n = n_pad // 16
    mesh = plsc.VectorSubcoreMesh(core_axis_name="c", subcore_axis_name="s", num_cores=2, num_subcores=16)

    @functools.partial(
        pl.kernel,
        out_type=jax.ShapeDtypeStruct((c, n_pad, 128), jnp.float32),
        mesh=mesh,
        scratch_types=[
            pltpu.VMEM((rows_per_tile, ew), jnp.int32),
            pltpu.VMEM((rows_per_tile, ew), jnp.int32),
            pltpu.VMEM((ew, 128), jnp.float32),
            pltpu.VMEM_SHARED((n_pad, 128), jnp.float32),
        ],
    )
    def k(g_hbm, src_hbm, dst_hbm, out_hbm, src_v, dst_v, rows_v, s_sh):
        cid = lax.axis_index("c")
        sid = lax.axis_index("s")
        er0 = sid * rows_per_tile
        pltpu.sync_copy(src_hbm.at[pl.ds(er0, rows_per_tile)], src_v)
        pltpu.sync_copy(dst_hbm.at[pl.ds(er0, rows_per_tile)], dst_v)
        nb = sid * rpt_n
        for chunk in range(c):

            @pl.when(chunk % 2 == cid)
            def _(chunk=chunk):
                g_c = g_hbm.at[chunk]
                pltpu.sync_copy(g_c.at[pl.ds(nb, rpt_n)],
                                s_sh.at[pl.ds(nb, rpt_n)])
                plsc.subcore_barrier()

                def body(j, carry):
                    pltpu.sync_copy(g_c.at[src_v.at[j]], rows_v)
                    pltpu.sync_copy(rows_v, s_sh.at[dst_v.at[j]], add=True)
                    return carry

                lax.fori_loop(0, rows_per_tile, body, 0)
                plsc.subcore_barrier()
                pltpu.sync_copy(s_sh.at[pl.ds(nb, rpt_n)],
                                out_hbm.at[chunk, pl.ds(nb, rpt_n)])
                plsc.subcore_barrier()

    return k(g, src2, dst2)


def kernel(x, edge_index, W1, b1, W2, b2, W3, b3):
    n = x.shape[0]
    e = edge_index.shape[1]
    n_pad = -(-n // 128) * 128
    src2 = edge_index[0].reshape(e // EW, EW)
    dst2 = edge_index[1].reshape(e // EW, EW)

    degS = _sc_deg(dst2, jnp.zeros((n_pad, 128), jnp.float32),
                   jnp.ones((EW, 128), jnp.float32))
    dinv16 = _dinv16(degS, n)

    h = x
    for W, b in ((W1, b1), (W2, b2), (W3, b3)):
        g = _matmul_g(h, W, dinv16, n_pad)
        S = _sc_edge(g, src2, dst2)
        h = _activate(S, dinv16, b.reshape(-1, 1, 128), n)
    return h


# speedup vs baseline: 1.9947x; 1.0296x over previous
import functools

import jax
import jax.numpy as jnp
from jax import lax
from jax.experimental import pallas as pl
from jax.experimental.pallas import tpu as pltpu
from jax.experimental.pallas import tpu_sc as plsc

EW = 125
MB = 2000


def _dinv16(degS, n):
    def body(d_ref, out_ref):
        out_ref[...] = 1.0 / jnp.sqrt(1.0 + d_ref[0, :, :16] + d_ref[1, :, :16])

    return pl.pallas_call(
        body,
        grid=(n // MB,),
        in_specs=[pl.BlockSpec((2, MB, 128), lambda m: (0, m, 0))],
        out_specs=pl.BlockSpec((MB, 16), lambda m: (m, 0)),
        out_shape=jax.ShapeDtypeStruct((n, 16), jnp.float32),
    )(degS)


def _sc_deg(dst2, zeros128, ones128):
    nrows, ew = dst2.shape
    rows_per_tile = nrows // 32
    n_pad = zeros128.shape[0]
    rpt_n = n_pad // 16
    mesh = plsc.VectorSubcoreMesh(core_axis_name="c", subcore_axis_name="s",
                                  num_cores=2, num_subcores=16)

    @functools.partial(
        pl.kernel,
        out_type=jax.ShapeDtypeStruct((2, n_pad, 128), jnp.float32),
        mesh=mesh,
        scratch_types=[
            pltpu.VMEM((rows_per_tile, ew), jnp.int32),
            pltpu.VMEM((ew, 128), jnp.float32),
            pltpu.VMEM_SHARED((n_pad, 128), jnp.float32),
        ],
    )
    def k(dst_hbm, z_hbm, ones_hbm, out_hbm, dst_v, ones_v, s_sh):
        cid = lax.axis_index("c")
        sid = lax.axis_index("s")
        er0 = (cid * 16 + sid) * rows_per_tile
        pltpu.sync_copy(dst_hbm.at[pl.ds(er0, rows_per_tile)], dst_v)
        pltpu.sync_copy(ones_hbm, ones_v)
        nb = sid * rpt_n
        pltpu.sync_copy(z_hbm.at[pl.ds(nb, rpt_n)], s_sh.at[pl.ds(nb, rpt_n)])
        plsc.subcore_barrier()

        def body(j, carry):
            pltpu.sync_copy(ones_v, s_sh.at[dst_v.at[j]], add=True)
            return carry

        lax.fori_loop(0, rows_per_tile, body, 0)
        plsc.subcore_barrier()
        pltpu.sync_copy(s_sh.at[pl.ds(nb, rpt_n)],
                        out_hbm.at[cid, pl.ds(nb, rpt_n)])

    return k(dst2, zeros128, ones128)


def _matmul_g(h, W, dinv16, n_pad):
    n, fin = h.shape
    fout = W.shape[1]
    cin, cout = fin // 128, fout // 128

    def body(h_ref, w_ref, dinv_ref, out_ref):
        r = jnp.dot(h_ref[...], w_ref[...],
                    preferred_element_type=jnp.float32) * dinv_ref[:, :1]
        out_ref[0] = r[:, :128]
        out_ref[1] = r[:, 128:]

    return pl.pallas_call(
        body,
        grid=(n // MB, cout // 2),
        in_specs=[
            pl.BlockSpec((MB, fin), lambda m, co: (m, 0)),
            pl.BlockSpec((fin, 256), lambda m, co: (0, co)),
            pl.BlockSpec((MB, 16), lambda m, co: (m, 0)),
        ],
        out_specs=pl.BlockSpec((2, MB, 128), lambda m, co: (co, m, 0)),
        out_shape=jax.ShapeDtypeStruct((cout, n_pad, 128), jnp.float32),
        compiler_params=pltpu.CompilerParams(
            dimension_semantics=("parallel", "parallel")),
    )(h, W, dinv16)


def _matmul_fused(S, dinv16, b2d, W, n, n_pad):
    cin = S.shape[0]
    fin = cin * 128
    fout = W.shape[1]
    cout = fout // 128

    def body(s_ref, dinv_ref, b_ref, w_ref, out_ref, h_s):
        co = pl.program_id(1)
        dinv = dinv_ref[:, :1]

        @pl.when(co == 0)
        def _():
            for ci in range(cin):
                h_s[:, ci * 128:(ci + 1) * 128] = jnp.tanh(
                    s_ref[ci] * dinv + b_ref[ci])

        r = jnp.dot(h_s[...], w_ref[...],
                    preferred_element_type=jnp.float32) * dinv
        out_ref[0] = r[:, :128]
        out_ref[1] = r[:, 128:]

    return pl.pallas_call(
        body,
        grid=(n // MB, cout // 2),
        in_specs=[
            pl.BlockSpec((cin, MB, 128), lambda m, co: (0, m, 0)),
            pl.BlockSpec((MB, 16), lambda m, co: (m, 0)),
            pl.BlockSpec((cin, 1, 128), lambda m, co: (0, 0, 0)),
            pl.BlockSpec((fin, 256), lambda m, co: (0, co)),
        ],
        out_specs=pl.BlockSpec((2, MB, 128), lambda m, co: (co, m, 0)),
        out_shape=jax.ShapeDtypeStruct((cout, n_pad, 128), jnp.float32),
        scratch_shapes=[pltpu.VMEM((MB, fin), jnp.float32)],
        compiler_params=pltpu.CompilerParams(
            dimension_semantics=("arbitrary", "arbitrary")),
    )(S, dinv16, b2d, W)


def _activate(S, dinv16, b2d, n):
    c = S.shape[0]

    def body(s_ref, dinv_ref, b_ref, out_ref):
        dinv = dinv_ref[:, :1]
        for ci in range(c):
            out_ref[:, ci * 128:(ci + 1) * 128] = jnp.tanh(
                s_ref[ci] * dinv + b_ref[ci])

    return pl.pallas_call(
        body,
        grid=(n // MB,),
        in_specs=[
            pl.BlockSpec((c, MB, 128), lambda m: (0, m, 0)),
            pl.BlockSpec((MB, 16), lambda m: (m, 0)),
            pl.BlockSpec((c, 1, 128), lambda m: (0, 0, 0)),
        ],
        out_specs=pl.BlockSpec((MB, c * 128), lambda m: (m, 0)),
        out_shape=jax.ShapeDtypeStruct((n, c * 128), jnp.float32),
    )(S, dinv16, b2d)


def _sc_edge(g, src2, dst2):
    c, n_pad, _ = g.shape
    nrows, ew = src2.shape
    rows_per_tile = nrows // 16
    rpt_n = n_pad // 16
    mesh = plsc.VectorSubcoreMesh(core_axis_name="c", subcore_axis_name="s", num_cores=2, num_subcores=16)

    @functools.partial(
        pl.kernel,
        out_type=jax.ShapeDtypeStruct((c, n_pad, 128), jnp.float32),
        mesh=mesh,
        scratch_types=[
            pltpu.VMEM((rows_per_tile, ew), jnp.int32),
            pltpu.VMEM((rows_per_tile, ew), jnp.int32),
            pltpu.VMEM((ew, 128), jnp.float32),
            pltpu.VMEM_SHARED((n_pad, 128), jnp.float32),
        ],
    )
    def k(g_hbm, src_hbm, dst_hbm, out_hbm, src_v, dst_v, rows_v, s_sh):
        cid = lax.axis_index("c")
        sid = lax.axis_index("s")
        er0 = sid * rows_per_tile
        pltpu.sync_copy(src_hbm.at[pl.ds(er0, rows_per_tile)], src_v)
        pltpu.sync_copy(dst_hbm.at[pl.ds(er0, rows_per_tile)], dst_v)
        nb = sid * rpt_n
        for chunk in range(c):

            @pl.when(chunk % 2 == cid)
            def _(chunk=chunk):
                g_c = g_hbm.at[chunk]
                pltpu.sync_copy(g_c.at[pl.ds(nb, rpt_n)],
                                s_sh.at[pl.ds(nb, rpt_n)])
                plsc.subcore_barrier()

                def body(j, carry):
                    pltpu.sync_copy(g_c.at[src_v.at[j]], rows_v)
                    pltpu.sync_copy(rows_v, s_sh.at[dst_v.at[j]], add=True)
                    return carry

                lax.fori_loop(0, rows_per_tile, body, 0)
                plsc.subcore_barrier()
                pltpu.sync_copy(s_sh.at[pl.ds(nb, rpt_n)],
                                out_hbm.at[chunk, pl.ds(nb, rpt_n)])
                plsc.subcore_barrier()

    return k(g, src2, dst2)


def kernel(x, edge_index, W1, b1, W2, b2, W3, b3):
    n = x.shape[0]
    e = edge_index.shape[1]
    n_pad = -(-n // 128) * 128
    src2 = edge_index[0].reshape(e // EW, EW)
    dst2 = edge_index[1].reshape(e // EW, EW)

    degS = _sc_deg(dst2, jnp.zeros((n_pad, 128), jnp.float32),
                   jnp.ones((EW, 128), jnp.float32))
    dinv16 = _dinv16(degS, n)

    g = _matmul_g(x, W1, dinv16, n_pad)
    S = _sc_edge(g, src2, dst2)
    for W, b_prev in ((W2, b1), (W3, b2)):
        g = _matmul_fused(S, dinv16, b_prev.reshape(-1, 1, 128), W, n, n_pad)
        S = _sc_edge(g, src2, dst2)
    return _activate(S, dinv16, b3.reshape(-1, 1, 128), n)


# speedup vs baseline: 1.9973x; 1.0013x over previous
import functools

import jax
import jax.numpy as jnp
from jax import lax
from jax.experimental import pallas as pl
from jax.experimental.pallas import tpu as pltpu
from jax.experimental.pallas import tpu_sc as plsc

EW = 125
MB = 2000


def _dinv16(degS, n):
    def body(d_ref, out_ref):
        out_ref[...] = 1.0 / jnp.sqrt(1.0 + d_ref[0, :, :16] + d_ref[1, :, :16])

    return pl.pallas_call(
        body,
        grid=(n // MB,),
        in_specs=[pl.BlockSpec((2, MB, 128), lambda m: (0, m, 0))],
        out_specs=pl.BlockSpec((MB, 16), lambda m: (m, 0)),
        out_shape=jax.ShapeDtypeStruct((n, 16), jnp.float32),
    )(degS)


def _sc_deg(dst2, zeros128, ones128):
    nrows, ew = dst2.shape
    rows_per_tile = nrows // 32
    n_pad = zeros128.shape[0]
    rpt_n = n_pad // 16
    mesh = plsc.VectorSubcoreMesh(core_axis_name="c", subcore_axis_name="s",
                                  num_cores=2, num_subcores=16)

    @functools.partial(
        pl.kernel,
        out_type=jax.ShapeDtypeStruct((2, n_pad, 128), jnp.float32),
        mesh=mesh,
        scratch_types=[
            pltpu.VMEM((rows_per_tile, ew), jnp.int32),
            pltpu.VMEM((ew, 128), jnp.float32),
            pltpu.VMEM_SHARED((n_pad, 128), jnp.float32),
        ],
    )
    def k(dst_hbm, z_hbm, ones_hbm, out_hbm, dst_v, ones_v, s_sh):
        cid = lax.axis_index("c")
        sid = lax.axis_index("s")
        er0 = (cid * 16 + sid) * rows_per_tile
        pltpu.sync_copy(dst_hbm.at[pl.ds(er0, rows_per_tile)], dst_v)
        pltpu.sync_copy(ones_hbm, ones_v)
        nb = sid * rpt_n
        pltpu.sync_copy(z_hbm.at[pl.ds(nb, rpt_n)], s_sh.at[pl.ds(nb, rpt_n)])
        plsc.subcore_barrier()

        def body(j, carry):
            pltpu.sync_copy(ones_v, s_sh.at[dst_v.at[j]], add=True)
            return carry

        lax.fori_loop(0, rows_per_tile, body, 0)
        plsc.subcore_barrier()
        pltpu.sync_copy(s_sh.at[pl.ds(nb, rpt_n)],
                        out_hbm.at[cid, pl.ds(nb, rpt_n)])

    return k(dst2, zeros128, ones128)


def _matmul_plain(h, W, n_pad):
    n, fin = h.shape
    fout = W.shape[1]
    cout = fout // 128

    def body(h_ref, w_ref, out_ref):
        r = jnp.dot(h_ref[...], w_ref[...], preferred_element_type=jnp.float32)
        out_ref[0] = r[:, :128]
        out_ref[1] = r[:, 128:]

    return pl.pallas_call(
        body,
        grid=(n // MB, cout // 2),
        in_specs=[
            pl.BlockSpec((MB, fin), lambda m, co: (m, 0)),
            pl.BlockSpec((fin, 256), lambda m, co: (0, co)),
        ],
        out_specs=pl.BlockSpec((2, MB, 128), lambda m, co: (co, m, 0)),
        out_shape=jax.ShapeDtypeStruct((cout, n_pad, 128), jnp.float32),
        compiler_params=pltpu.CompilerParams(
            dimension_semantics=("parallel", "parallel")),
    )(h, W)


def _scale(hW, dinv16, n):
    c = hW.shape[0]
    n_pad = hW.shape[1]

    def body(h_ref, dinv_ref, out_ref):
        dinv = dinv_ref[:, :1]
        for ci in range(c):
            out_ref[ci] = h_ref[ci] * dinv

    return pl.pallas_call(
        body,
        grid=(n // MB,),
        in_specs=[
            pl.BlockSpec((c, MB, 128), lambda m: (0, m, 0)),
            pl.BlockSpec((MB, 16), lambda m: (m, 0)),
        ],
        out_specs=pl.BlockSpec((c, MB, 128), lambda m: (0, m, 0)),
        out_shape=jax.ShapeDtypeStruct((c, n_pad, 128), jnp.float32),
    )(hW, dinv16)


def _matmul_fused(S, dinv16, b2d, W, n, n_pad):
    cin = S.shape[0]
    fin = cin * 128
    fout = W.shape[1]
    cout = fout // 128

    def body(s_ref, dinv_ref, b_ref, w_ref, out_ref, h_s):
        co = pl.program_id(1)
        dinv = dinv_ref[:, :1]

        @pl.when(co == 0)
        def _():
            for ci in range(cin):
                h_s[:, ci * 128:(ci + 1) * 128] = jnp.tanh(
                    s_ref[ci] * dinv + b_ref[ci])

        r = jnp.dot(h_s[...], w_ref[...],
                    preferred_element_type=jnp.float32) * dinv
        out_ref[0] = r[:, :128]
        out_ref[1] = r[:, 128:]

    return pl.pallas_call(
        body,
        grid=(n // MB, cout // 2),
        in_specs=[
            pl.BlockSpec((cin, MB, 128), lambda m, co: (0, m, 0)),
            pl.BlockSpec((MB, 16), lambda m, co: (m, 0)),
            pl.BlockSpec((cin, 1, 128), lambda m, co: (0, 0, 0)),
            pl.BlockSpec((fin, 256), lambda m, co: (0, co)),
        ],
        out_specs=pl.BlockSpec((2, MB, 128), lambda m, co: (co, m, 0)),
        out_shape=jax.ShapeDtypeStruct((cout, n_pad, 128), jnp.float32),
        scratch_shapes=[pltpu.VMEM((MB, fin), jnp.float32)],
        compiler_params=pltpu.CompilerParams(
            dimension_semantics=("arbitrary", "arbitrary")),
    )(S, dinv16, b2d, W)


def _activate(S, dinv16, b2d, n):
    c = S.shape[0]

    def body(s_ref, dinv_ref, b_ref, out_ref):
        dinv = dinv_ref[:, :1]
        for ci in range(c):
            out_ref[:, ci * 128:(ci + 1) * 128] = jnp.tanh(
                s_ref[ci] * dinv + b_ref[ci])

    return pl.pallas_call(
        body,
        grid=(n // MB,),
        in_specs=[
            pl.BlockSpec((c, MB, 128), lambda m: (0, m, 0)),
            pl.BlockSpec((MB, 16), lambda m: (m, 0)),
            pl.BlockSpec((c, 1, 128), lambda m: (0, 0, 0)),
        ],
        out_specs=pl.BlockSpec((MB, c * 128), lambda m: (m, 0)),
        out_shape=jax.ShapeDtypeStruct((n, c * 128), jnp.float32),
    )(S, dinv16, b2d)


def _sc_edge(g, src2, dst2):
    c, n_pad, _ = g.shape
    nrows, ew = src2.shape
    rows_per_tile = nrows // 16
    rpt_n = n_pad // 16
    mesh = plsc.VectorSubcoreMesh(core_axis_name="c", subcore_axis_name="s", num_cores=2, num_subcores=16)

    @functools.partial(
        pl.kernel,
        out_type=jax.ShapeDtypeStruct((c, n_pad, 128), jnp.float32),
        mesh=mesh,
        scratch_types=[
            pltpu.VMEM((rows_per_tile, ew), jnp.int32),
            pltpu.VMEM((rows_per_tile, ew), jnp.int32),
            pltpu.VMEM((ew, 128), jnp.float32),
            pltpu.VMEM_SHARED((n_pad, 128), jnp.float32),
        ],
    )
    def k(g_hbm, src_hbm, dst_hbm, out_hbm, src_v, dst_v, rows_v, s_sh):
        cid = lax.axis_index("c")
        sid = lax.axis_index("s")
        er0 = sid * rows_per_tile
        pltpu.sync_copy(src_hbm.at[pl.ds(er0, rows_per_tile)], src_v)
        pltpu.sync_copy(dst_hbm.at[pl.ds(er0, rows_per_tile)], dst_v)
        nb = sid * rpt_n
        for chunk in range(c):

            @pl.when(chunk % 2 == cid)
            def _(chunk=chunk):
                g_c = g_hbm.at[chunk]
                pltpu.sync_copy(g_c.at[pl.ds(nb, rpt_n)],
                                s_sh.at[pl.ds(nb, rpt_n)])
                plsc.subcore_barrier()

                def body(j, carry):
                    pltpu.sync_copy(g_c.at[src_v.at[j]], rows_v)
                    pltpu.sync_copy(rows_v, s_sh.at[dst_v.at[j]], add=True)
                    return carry

                lax.fori_loop(0, rows_per_tile, body, 0)
                plsc.subcore_barrier()
                pltpu.sync_copy(s_sh.at[pl.ds(nb, rpt_n)],
                                out_hbm.at[chunk, pl.ds(nb, rpt_n)])
                plsc.subcore_barrier()

    return k(g, src2, dst2)


def kernel(x, edge_index, W1, b1, W2, b2, W3, b3):
    n = x.shape[0]
    e = edge_index.shape[1]
    n_pad = -(-n // 128) * 128
    src2 = edge_index[0].reshape(e // EW, EW)
    dst2 = edge_index[1].reshape(e // EW, EW)

    degS = _sc_deg(dst2, jnp.zeros((n_pad, 128), jnp.float32),
                   jnp.ones((EW, 128), jnp.float32))
    dinv16 = _dinv16(degS, n)

    g = _scale(_matmul_plain(x, W1, n_pad), dinv16, n)
    S = _sc_edge(g, src2, dst2)
    for W, b_prev in ((W2, b1), (W3, b2)):
        g = _matmul_fused(S, dinv16, b_prev.reshape(-1, 1, 128), W, n, n_pad)
        S = _sc_edge(g, src2, dst2)
    return _activate(S, dinv16, b3.reshape(-1, 1, 128), n)


# speedup vs baseline: 2.5319x; 1.2677x over previous
import functools

import jax
import jax.numpy as jnp
from jax import lax
from jax.experimental import pallas as pl
from jax.experimental.pallas import tpu as pltpu
from jax.experimental.pallas import tpu_sc as plsc

EW = 125
MB = 2000


def _dinv16(degS, n):
    def body(d_ref, out_ref):
        out_ref[...] = 1.0 / jnp.sqrt(1.0 + d_ref[0, :, :16] + d_ref[1, :, :16])

    return pl.pallas_call(
        body,
        grid=(n // MB,),
        in_specs=[pl.BlockSpec((2, MB, 128), lambda m: (0, m, 0))],
        out_specs=pl.BlockSpec((MB, 16), lambda m: (m, 0)),
        out_shape=jax.ShapeDtypeStruct((n, 16), jnp.float32),
    )(degS)


def _sc_deg(dst2, zeros128, ones128):
    nrows, ew = dst2.shape
    rows_per_tile = nrows // 32
    n_pad = zeros128.shape[0]
    rpt_n = n_pad // 16
    mesh = plsc.VectorSubcoreMesh(core_axis_name="c", subcore_axis_name="s",
                                  num_cores=2, num_subcores=16)

    @functools.partial(
        pl.kernel,
        out_type=jax.ShapeDtypeStruct((2, n_pad, 128), jnp.float32),
        mesh=mesh,
        scratch_types=[
            pltpu.VMEM((rows_per_tile, ew), jnp.int32),
            pltpu.VMEM((ew, 128), jnp.float32),
            pltpu.VMEM_SHARED((n_pad, 128), jnp.float32),
        ],
    )
    def k(dst_hbm, z_hbm, ones_hbm, out_hbm, dst_v, ones_v, s_sh):
        cid = lax.axis_index("c")
        sid = lax.axis_index("s")
        er0 = (cid * 16 + sid) * rows_per_tile
        pltpu.sync_copy(dst_hbm.at[pl.ds(er0, rows_per_tile)], dst_v)
        pltpu.sync_copy(ones_hbm, ones_v)
        nb = sid * rpt_n
        pltpu.sync_copy(z_hbm.at[pl.ds(nb, rpt_n)], s_sh.at[pl.ds(nb, rpt_n)])
        plsc.subcore_barrier()

        def body(j, carry):
            pltpu.sync_copy(ones_v, s_sh.at[dst_v.at[j]], add=True)
            return carry

        lax.fori_loop(0, rows_per_tile, body, 0)
        plsc.subcore_barrier()
        pltpu.sync_copy(s_sh.at[pl.ds(nb, rpt_n)],
                        out_hbm.at[cid, pl.ds(nb, rpt_n)])

    return k(dst2, zeros128, ones128)


def _matmul_plain(h, W, n_pad):
    n, fin = h.shape
    fout = W.shape[1]
    cout = fout // 128

    def body(h_ref, w_ref, out_ref):
        r = jnp.dot(h_ref[...], w_ref[...], preferred_element_type=jnp.float32)
        out_ref[0] = r[:, :128]
        out_ref[1] = r[:, 128:]

    return pl.pallas_call(
        body,
        grid=(n // MB, cout // 2),
        in_specs=[
            pl.BlockSpec((MB, fin), lambda m, co: (m, 0)),
            pl.BlockSpec((fin, 256), lambda m, co: (0, co)),
        ],
        out_specs=pl.BlockSpec((2, MB, 128), lambda m, co: (co, m, 0)),
        out_shape=jax.ShapeDtypeStruct((cout, n_pad, 128), jnp.float32),
        compiler_params=pltpu.CompilerParams(
            dimension_semantics=("parallel", "parallel")),
    )(h, W)


def _scale(hW, dinv16, n):
    c = hW.shape[0]
    n_pad = hW.shape[1]

    def body(h_ref, dinv_ref, out_ref):
        dinv = dinv_ref[:, :1]
        for ci in range(c):
            out_ref[ci] = h_ref[ci] * dinv

    return pl.pallas_call(
        body,
        grid=(n // MB,),
        in_specs=[
            pl.BlockSpec((c, MB, 128), lambda m: (0, m, 0)),
            pl.BlockSpec((MB, 16), lambda m: (m, 0)),
        ],
        out_specs=pl.BlockSpec((c, MB, 128), lambda m: (0, m, 0)),
        out_shape=jax.ShapeDtypeStruct((c, n_pad, 128), jnp.float32),
    )(hW, dinv16)


def _matmul_fused(S, dinv16, b2d, W, n, n_pad):
    cin = S.shape[0]
    fin = cin * 128
    fout = W.shape[1]
    cout = fout // 128

    def body(s_ref, dinv_ref, b_ref, w_ref, out_ref, h_s):
        co = pl.program_id(1)
        dinv = dinv_ref[:, :1]

        @pl.when(co == 0)
        def _():
            for ci in range(cin):
                h_s[:, ci * 128:(ci + 1) * 128] = jnp.tanh(
                    s_ref[ci] * dinv + b_ref[ci])

        r = jnp.dot(h_s[...], w_ref[...],
                    preferred_element_type=jnp.float32) * dinv
        out_ref[0] = r[:, :128]
        out_ref[1] = r[:, 128:]

    return pl.pallas_call(
        body,
        grid=(n // MB, cout // 2),
        in_specs=[
            pl.BlockSpec((cin, MB, 128), lambda m, co: (0, m, 0)),
            pl.BlockSpec((MB, 16), lambda m, co: (m, 0)),
            pl.BlockSpec((cin, 1, 128), lambda m, co: (0, 0, 0)),
            pl.BlockSpec((fin, 256), lambda m, co: (0, co)),
        ],
        out_specs=pl.BlockSpec((2, MB, 128), lambda m, co: (co, m, 0)),
        out_shape=jax.ShapeDtypeStruct((cout, n_pad, 128), jnp.float32),
        scratch_shapes=[pltpu.VMEM((MB, fin), jnp.float32)],
        compiler_params=pltpu.CompilerParams(
            dimension_semantics=("arbitrary", "arbitrary")),
    )(S, dinv16, b2d, W)


def _activate(S, dinv16, b2d, n):
    c = S.shape[0]

    def body(s_ref, dinv_ref, b_ref, out_ref):
        dinv = dinv_ref[:, :1]
        for ci in range(c):
            out_ref[:, ci * 128:(ci + 1) * 128] = jnp.tanh(
                s_ref[ci] * dinv + b_ref[ci])

    return pl.pallas_call(
        body,
        grid=(n // MB,),
        in_specs=[
            pl.BlockSpec((c, MB, 128), lambda m: (0, m, 0)),
            pl.BlockSpec((MB, 16), lambda m: (m, 0)),
            pl.BlockSpec((c, 1, 128), lambda m: (0, 0, 0)),
        ],
        out_specs=pl.BlockSpec((MB, c * 128), lambda m: (m, 0)),
        out_shape=jax.ShapeDtypeStruct((n, c * 128), jnp.float32),
    )(S, dinv16, b2d)


def _sc_edge(g, src2, dst2):
    c, n_pad, _ = g.shape
    nrows, ew = src2.shape
    rows_per_tile = nrows // 16
    rpt_n = n_pad // 16
    mesh = plsc.VectorSubcoreMesh(core_axis_name="c", subcore_axis_name="s", num_cores=2, num_subcores=16)

    rows_per_phase = rows_per_tile // 2
    scratch_types = [
            pltpu.VMEM((rows_per_phase, ew), jnp.int32),
            pltpu.VMEM((rows_per_phase, ew), jnp.int32),
            pltpu.VMEM((ew, 128), jnp.float32),
            pltpu.VMEM((ew, 128), jnp.float32),
            pltpu.VMEM_SHARED((n_pad, 128), jnp.float32),
            pltpu.SemaphoreType.DMA,
            pltpu.SemaphoreType.DMA,
    ]

    @functools.partial(
        pl.kernel,
        out_type=jax.ShapeDtypeStruct((c, n_pad, 128), jnp.float32),
        mesh=mesh,
        scratch_types=scratch_types,
    )
    def k(g_hbm, src_hbm, dst_hbm, out_hbm, src_v, dst_v, rows0_v, rows1_v,
          s_sh, sem0, sem1):
        cid = lax.axis_index("c")
        sid = lax.axis_index("s")
        er0 = sid * rows_per_tile
        nb = sid * rpt_n
        for chunk in range(c):

            @pl.when(chunk % 2 == cid)
            def _(chunk=chunk):
                g_c = g_hbm.at[chunk]
                pltpu.sync_copy(g_c.at[pl.ds(nb, rpt_n)],
                                s_sh.at[pl.ds(nb, rpt_n)])
                plsc.subcore_barrier()

                for phase in range(2):
                    p0 = er0 + phase * rows_per_phase
                    pltpu.sync_copy(src_hbm.at[pl.ds(p0, rows_per_phase)],
                                    src_v)
                    pltpu.sync_copy(dst_hbm.at[pl.ds(p0, rows_per_phase)],
                                    dst_v)

                    def body(i, carry):
                        j = 2 * i

                        @pl.when(i > 0)
                        def _():
                            pltpu.make_async_copy(
                                rows0_v, s_sh.at[dst_v.at[j]], sem0).wait()

                        pltpu.sync_copy(g_c.at[src_v.at[j]], rows0_v)
                        pltpu.async_copy(rows0_v, s_sh.at[dst_v.at[j]],
                                         sem0, add=True)

                        @pl.when(i > 0)
                        def _():
                            pltpu.make_async_copy(
                                rows1_v, s_sh.at[dst_v.at[j + 1]],
                                sem1).wait()

                        pltpu.sync_copy(g_c.at[src_v.at[j + 1]], rows1_v)
                        pltpu.async_copy(rows1_v, s_sh.at[dst_v.at[j + 1]],
                                         sem1, add=True)
                        return carry

                    lax.fori_loop(0, rows_per_phase // 2, body, 0)
                    pltpu.make_async_copy(rows0_v, s_sh.at[dst_v.at[0]],
                                          sem0).wait()
                    pltpu.make_async_copy(rows1_v, s_sh.at[dst_v.at[1]],
                                          sem1).wait()
                plsc.subcore_barrier()
                pltpu.sync_copy(s_sh.at[pl.ds(nb, rpt_n)],
                                out_hbm.at[chunk, pl.ds(nb, rpt_n)])
                plsc.subcore_barrier()

    return k(g, src2, dst2)


def kernel(x, edge_index, W1, b1, W2, b2, W3, b3):
    n = x.shape[0]
    e = edge_index.shape[1]
    n_pad = -(-n // 128) * 128
    src2 = edge_index[0].reshape(e // EW, EW)
    dst2 = edge_index[1].reshape(e // EW, EW)

    degS = _sc_deg(dst2, jnp.zeros((n_pad, 128), jnp.float32),
                   jnp.ones((EW, 128), jnp.float32))
    dinv16 = _dinv16(degS, n)

    g = _scale(_matmul_plain(x, W1, n_pad), dinv16, n)
    S = _sc_edge(g, src2, dst2)
    for W, b_prev in ((W2, b1), (W3, b2)):
        g = _matmul_fused(S, dinv16, b_prev.reshape(-1, 1, 128), W, n, n_pad)
        S = _sc_edge(g, src2, dst2)
    return _activate(S, dinv16, b3.reshape(-1, 1, 128), n)


# speedup vs baseline: 2.5500x; 1.0072x over previous
import functools

import jax
import jax.numpy as jnp
from jax import lax
from jax.experimental import pallas as pl
from jax.experimental.pallas import tpu as pltpu
from jax.experimental.pallas import tpu_sc as plsc

EW = 125
MB = 2000


def _sc_deg(dst2, zeros128, ones128):
    nrows, ew = dst2.shape
    rows_per_tile = nrows // 32
    n_pad = zeros128.shape[0]
    rpt_n = n_pad // 16
    mesh = plsc.VectorSubcoreMesh(core_axis_name="c", subcore_axis_name="s",
                                  num_cores=2, num_subcores=16)

    @functools.partial(
        pl.kernel,
        out_type=jax.ShapeDtypeStruct((2, n_pad, 128), jnp.float32),
        mesh=mesh,
        scratch_types=[
            pltpu.VMEM((rows_per_tile, ew), jnp.int32),
            pltpu.VMEM((ew, 128), jnp.float32),
            pltpu.VMEM_SHARED((n_pad, 128), jnp.float32),
            pltpu.SemaphoreType.DMA,
        ],
    )
    def k(dst_hbm, z_hbm, ones_hbm, out_hbm, dst_v, ones_v, s_sh, sem0):
        cid = lax.axis_index("c")
        sid = lax.axis_index("s")
        er0 = (cid * 16 + sid) * rows_per_tile
        pltpu.sync_copy(dst_hbm.at[pl.ds(er0, rows_per_tile)], dst_v)
        pltpu.sync_copy(ones_hbm, ones_v)
        nb = sid * rpt_n
        pltpu.sync_copy(z_hbm.at[pl.ds(nb, rpt_n)], s_sh.at[pl.ds(nb, rpt_n)])
        plsc.subcore_barrier()

        def body(j, carry):
            pltpu.async_copy(ones_v, s_sh.at[dst_v.at[j]], sem0, add=True)
            return carry

        lax.fori_loop(0, rows_per_tile, body, 0)

        def drain(j, carry):
            pltpu.make_async_copy(ones_v, s_sh.at[dst_v.at[0]], sem0).wait()
            return carry

        lax.fori_loop(0, rows_per_tile, drain, 0)
        plsc.subcore_barrier()
        pltpu.sync_copy(s_sh.at[pl.ds(nb, rpt_n)],
                        out_hbm.at[cid, pl.ds(nb, rpt_n)])

    return k(dst2, zeros128, ones128)


def _matmul_plain(h, W, n_pad):
    n, fin = h.shape
    fout = W.shape[1]
    cout = fout // 128

    def body(h_ref, w_ref, out_ref):
        r = jnp.dot(h_ref[...], w_ref[...], preferred_element_type=jnp.float32)
        out_ref[0] = r[:, :128]
        out_ref[1] = r[:, 128:]

    return pl.pallas_call(
        body,
        grid=(n // MB, cout // 2),
        in_specs=[
            pl.BlockSpec((MB, fin), lambda m, co: (m, 0)),
            pl.BlockSpec((fin, 256), lambda m, co: (0, co)),
        ],
        out_specs=pl.BlockSpec((2, MB, 128), lambda m, co: (co, m, 0)),
        out_shape=jax.ShapeDtypeStruct((cout, n_pad, 128), jnp.float32),
        compiler_params=pltpu.CompilerParams(
            dimension_semantics=("parallel", "parallel")),
    )(h, W)


def _scale_dinv(hW, degS, n):
    c = hW.shape[0]
    n_pad = hW.shape[1]

    def body(h_ref, d_ref, out_ref, dinv_ref):
        dinv = 1.0 / jnp.sqrt(1.0 + d_ref[0, :, :16] + d_ref[1, :, :16])
        dinv_ref[...] = dinv
        for ci in range(c):
            out_ref[ci] = h_ref[ci] * dinv[:, :1]

    return pl.pallas_call(
        body,
        grid=(n // MB,),
        in_specs=[
            pl.BlockSpec((c, MB, 128), lambda m: (0, m, 0)),
            pl.BlockSpec((2, MB, 128), lambda m: (0, m, 0)),
        ],
        out_specs=(
            pl.BlockSpec((c, MB, 128), lambda m: (0, m, 0)),
            pl.BlockSpec((MB, 16), lambda m: (m, 0)),
        ),
        out_shape=(
            jax.ShapeDtypeStruct((c, n_pad, 128), jnp.float32),
            jax.ShapeDtypeStruct((n, 16), jnp.float32),
        ),
    )(hW, degS)


def _matmul_fused(S, dinv16, b2d, W, n, n_pad):
    cin = S.shape[0]
    fin = cin * 128
    fout = W.shape[1]
    cout = fout // 128

    def body(s_ref, dinv_ref, b_ref, w_ref, out_ref, h_s):
        co = pl.program_id(1)
        dinv = dinv_ref[:, :1]

        @pl.when(co == 0)
        def _():
            for ci in range(cin):
                h_s[:, ci * 128:(ci + 1) * 128] = jnp.tanh(
                    s_ref[ci] * dinv + b_ref[ci])

        r = jnp.dot(h_s[...], w_ref[...],
                    preferred_element_type=jnp.float32) * dinv
        out_ref[0] = r[:, :128]
        out_ref[1] = r[:, 128:]

    return pl.pallas_call(
        body,
        grid=(n // MB, cout // 2),
        in_specs=[
            pl.BlockSpec((cin, MB, 128), lambda m, co: (0, m, 0)),
            pl.BlockSpec((MB, 16), lambda m, co: (m, 0)),
            pl.BlockSpec((cin, 1, 128), lambda m, co: (0, 0, 0)),
            pl.BlockSpec((fin, 256), lambda m, co: (0, co)),
        ],
        out_specs=pl.BlockSpec((2, MB, 128), lambda m, co: (co, m, 0)),
        out_shape=jax.ShapeDtypeStruct((cout, n_pad, 128), jnp.float32),
        scratch_shapes=[pltpu.VMEM((MB, fin), jnp.float32)],
        compiler_params=pltpu.CompilerParams(
            dimension_semantics=("arbitrary", "arbitrary")),
    )(S, dinv16, b2d, W)


def _activate(S, dinv16, b2d, n):
    c = S.shape[0]

    def body(s_ref, dinv_ref, b_ref, out_ref):
        dinv = dinv_ref[:, :1]
        for ci in range(c):
            out_ref[:, ci * 128:(ci + 1) * 128] = jnp.tanh(
                s_ref[ci] * dinv + b_ref[ci])

    return pl.pallas_call(
        body,
        grid=(n // MB,),
        in_specs=[
            pl.BlockSpec((c, MB, 128), lambda m: (0, m, 0)),
            pl.BlockSpec((MB, 16), lambda m: (m, 0)),
            pl.BlockSpec((c, 1, 128), lambda m: (0, 0, 0)),
        ],
        out_specs=pl.BlockSpec((MB, c * 128), lambda m: (m, 0)),
        out_shape=jax.ShapeDtypeStruct((n, c * 128), jnp.float32),
    )(S, dinv16, b2d)


def _sc_edge(g, src2, dst2):
    c, n_pad, _ = g.shape
    nrows, ew = src2.shape
    rows_per_tile = nrows // 16
    rpt_n = n_pad // 16
    mesh = plsc.VectorSubcoreMesh(core_axis_name="c", subcore_axis_name="s", num_cores=2, num_subcores=16)

    rows_per_phase = rows_per_tile // 2
    scratch_types = [
            pltpu.VMEM((rows_per_phase, ew), jnp.int32),
            pltpu.VMEM((rows_per_phase, ew), jnp.int32),
            pltpu.VMEM((ew, 128), jnp.float32),
            pltpu.VMEM((ew, 128), jnp.float32),
            pltpu.VMEM_SHARED((n_pad, 128), jnp.float32),
            pltpu.SemaphoreType.DMA,
            pltpu.SemaphoreType.DMA,
    ]

    @functools.partial(
        pl.kernel,
        out_type=jax.ShapeDtypeStruct((c, n_pad, 128), jnp.float32),
        mesh=mesh,
        scratch_types=scratch_types,
    )
    def k(g_hbm, src_hbm, dst_hbm, out_hbm, src_v, dst_v, rows0_v, rows1_v,
          s_sh, sem0, sem1):
        cid = lax.axis_index("c")
        sid = lax.axis_index("s")
        er0 = sid * rows_per_tile
        nb = sid * rpt_n
        for chunk in range(c):

            @pl.when(chunk % 2 == cid)
            def _(chunk=chunk):
                g_c = g_hbm.at[chunk]
                pltpu.sync_copy(g_c.at[pl.ds(nb, rpt_n)],
                                s_sh.at[pl.ds(nb, rpt_n)])
                plsc.subcore_barrier()

                for phase in range(2):
                    p0 = er0 + phase * rows_per_phase
                    pltpu.sync_copy(src_hbm.at[pl.ds(p0, rows_per_phase)],
                                    src_v)
                    pltpu.sync_copy(dst_hbm.at[pl.ds(p0, rows_per_phase)],
                                    dst_v)

                    def body(i, carry):
                        j = 2 * i

                        @pl.when(i > 0)
                        def _():
                            pltpu.make_async_copy(
                                rows0_v, s_sh.at[dst_v.at[j]], sem0).wait()

                        pltpu.sync_copy(g_c.at[src_v.at[j]], rows0_v)
                        pltpu.async_copy(rows0_v, s_sh.at[dst_v.at[j]],
                                         sem0, add=True)

                        @pl.when(i > 0)
                        def _():
                            pltpu.make_async_copy(
                                rows1_v, s_sh.at[dst_v.at[j + 1]],
                                sem1).wait()

                        pltpu.sync_copy(g_c.at[src_v.at[j + 1]], rows1_v)
                        pltpu.async_copy(rows1_v, s_sh.at[dst_v.at[j + 1]],
                                         sem1, add=True)
                        return carry

                    lax.fori_loop(0, rows_per_phase // 2, body, 0)
                    pltpu.make_async_copy(rows0_v, s_sh.at[dst_v.at[0]],
                                          sem0).wait()
                    pltpu.make_async_copy(rows1_v, s_sh.at[dst_v.at[1]],
                                          sem1).wait()
                plsc.subcore_barrier()
                pltpu.sync_copy(s_sh.at[pl.ds(nb, rpt_n)],
                                out_hbm.at[chunk, pl.ds(nb, rpt_n)])
                plsc.subcore_barrier()

    return k(g, src2, dst2)


def kernel(x, edge_index, W1, b1, W2, b2, W3, b3):
    n = x.shape[0]
    e = edge_index.shape[1]
    n_pad = -(-n // 128) * 128
    src2 = edge_index[0].reshape(e // EW, EW)
    dst2 = edge_index[1].reshape(e // EW, EW)

    degS = _sc_deg(dst2, jnp.zeros((n_pad, 128), jnp.float32),
                   jnp.ones((EW, 128), jnp.float32))
    g, dinv16 = _scale_dinv(_matmul_plain(x, W1, n_pad), degS, n)
    S = _sc_edge(g, src2, dst2)
    for W, b_prev in ((W2, b1), (W3, b2)):
        g = _matmul_fused(S, dinv16, b_prev.reshape(-1, 1, 128), W, n, n_pad)
        S = _sc_edge(g, src2, dst2)
    return _activate(S, dinv16, b3.reshape(-1, 1, 128), n)
